# baseline (speedup 1.0000x reference)
import functools

import jax
import jax.numpy as jnp
from jax import lax
from jax.experimental import pallas as pl
from jax.experimental.pallas import tpu as pltpu
from jax.experimental.pallas import tpu_sc as plsc

_B = 4096
_D = 64
_ROW_BLK = 256
_L = 16
_NLINE = 7813


def _sc_gather(targets, emb_table, blines):
    info = plsc.get_sparse_core_info()
    nc, ns = info.num_cores, info.num_subcores
    nw = nc * ns
    bpw = _B // nw

    mesh = plsc.VectorSubcoreMesh(core_axis_name="c", subcore_axis_name="s")

    @functools.partial(
        pl.kernel,
        mesh=mesh,
        compiler_params=pltpu.CompilerParams(
            use_tc_tiling_on_sc=False, needs_layout_passes=False),
        out_type=[
            jax.ShapeDtypeStruct((_B, _D), jnp.float32),
            jax.ShapeDtypeStruct((_B, 128), jnp.float32),
        ],
        scratch_types=[
            pltpu.VMEM((bpw,), jnp.int32),
            pltpu.VMEM((bpw,), jnp.int32),
            pltpu.VMEM((bpw, _D), jnp.float32),
            pltpu.VMEM((bpw, 128), jnp.float32),
            pltpu.SemaphoreType.DMA,
            pltpu.SemaphoreType.DMA,
        ],
    )
    def gather_kernel(tgt_hbm, emb_hbm, blines_hbm, rows_out, blines_out,
                      idx_v, line_v, rows_v, bl_v, sem_e, sem_b):
        wid = lax.axis_index("s") * nc + lax.axis_index("c")
        base = wid * bpw
        pltpu.sync_copy(tgt_hbm.at[pl.ds(base, bpw)], idx_v)
        for k in range(bpw // _L):
            sl = pl.ds(k * _L, _L)
            line_v[sl] = lax.shift_right_logical(idx_v[sl], 7)
        ce = pltpu.async_copy(emb_hbm.at[idx_v], rows_v, sem_e)
        cb = pltpu.async_copy(blines_hbm.at[line_v], bl_v, sem_b)
        ce.wait()
        cb.wait()
        pltpu.sync_copy(rows_v, rows_out.at[pl.ds(base, bpw)])
        pltpu.sync_copy(bl_v, blines_out.at[pl.ds(base, bpw)])

    return gather_kernel(targets, emb_table, blines)


def _dot_body(u_ref, g_ref, dot_ref):
    dot_ref[...] = jnp.sum(u_ref[...] * g_ref[...], axis=1)[None, :]


def _bcast_body(tb_ref, bl_ref, dot_ref, out_ref):
    tsub = tb_ref[...].reshape(_ROW_BLK) & 127
    sel = jnp.where(
        lax.broadcasted_iota(jnp.int32, (_ROW_BLK, 128), 1) == tsub[:, None],
        bl_ref[...], 0.0)
    bias_blk = jnp.sum(sel, axis=1)
    out_ref[...] = bias_blk[:, None] + dot_ref[...]


def kernel(user_representations, targets, emb_table, bias_table):
    targets = targets.astype(jnp.int32)
    blines = jnp.pad(bias_table.reshape(-1),
                     (0, _NLINE * 128 - bias_table.shape[0])).reshape(_NLINE, 128)
    g, btiles = _sc_gather(targets, emb_table, blines)
    dot = pl.pallas_call(
        _dot_body,
        out_shape=jax.ShapeDtypeStruct((1, _B), jnp.float32),
    )(user_representations, g)
    tgt3d = targets.reshape(_B // _ROW_BLK, 1, _ROW_BLK)
    return pl.pallas_call(
        _bcast_body,
        grid=(_B // _ROW_BLK,),
        in_specs=[
            pl.BlockSpec((1, 1, _ROW_BLK), lambda i: (i, 0, 0)),
            pl.BlockSpec((_ROW_BLK, 128), lambda i: (i, 0)),
            pl.BlockSpec((1, _B), lambda i: (0, 0)),
        ],
        out_specs=pl.BlockSpec((_ROW_BLK, _B), lambda i: (i, 0)),
        out_shape=jax.ShapeDtypeStruct((_B, _B), jnp.float32),
    )(tgt3d, btiles, dot)


# speedup vs baseline: 1.0023x; 1.0023x over previous
import functools

import jax
import jax.numpy as jnp
from jax import lax
from jax.experimental import pallas as pl
from jax.experimental.pallas import tpu as pltpu
from jax.experimental.pallas import tpu_sc as plsc

_B = 4096
_D = 64
_ROW_BLK = 256
_L = 16


def _sc_gather(targets, emb_table, bwords):
    info = plsc.get_sparse_core_info()
    nc, ns = info.num_cores, info.num_subcores
    nw = nc * ns
    bpw = _B // nw

    mesh = plsc.VectorSubcoreMesh(core_axis_name="c", subcore_axis_name="s")

    @functools.partial(
        pl.kernel,
        mesh=mesh,
        compiler_params=pltpu.CompilerParams(
            use_tc_tiling_on_sc=False, needs_layout_passes=False),
        out_type=[
            jax.ShapeDtypeStruct((_B, _D), jnp.float32),
            jax.ShapeDtypeStruct((_B, _L), jnp.float32),
        ],
        scratch_types=[
            pltpu.VMEM((bpw,), jnp.int32),
            pltpu.VMEM((bpw,), jnp.int32),
            pltpu.VMEM((bpw, _D), jnp.float32),
            pltpu.VMEM((bpw, _L), jnp.float32),
            pltpu.SemaphoreType.DMA,
            pltpu.SemaphoreType.DMA,
        ],
    )
    def gather_kernel(tgt_hbm, emb_hbm, bwords_hbm, rows_out, bwords_out,
                      idx_v, word_v, rows_v, bw_v, sem_e, sem_b):
        wid = lax.axis_index("s") * nc + lax.axis_index("c")
        base = wid * bpw
        pltpu.sync_copy(tgt_hbm.at[pl.ds(base, bpw)], idx_v)
        for k in range(bpw // _L):
            sl = pl.ds(k * _L, _L)
            word_v[sl] = lax.shift_right_logical(idx_v[sl], 4)
        ce = pltpu.async_copy(emb_hbm.at[idx_v], rows_v, sem_e)
        cb = pltpu.async_copy(bwords_hbm.at[word_v], bw_v, sem_b)
        ce.wait()
        cb.wait()
        pltpu.sync_copy(rows_v, rows_out.at[pl.ds(base, bpw)])
        pltpu.sync_copy(bw_v, bwords_out.at[pl.ds(base, bpw)])

    return gather_kernel(targets, emb_table, bwords)


def _bcast_body(tb_ref, bw_ref, u_ref, g_ref, out_ref, dot_scr):
    @pl.when(pl.program_id(0) == 0)
    def _():
        dot_scr[...] = jnp.sum(u_ref[...] * g_ref[...], axis=1)[None, :]

    tsub = tb_ref[...].reshape(_ROW_BLK) & (_L - 1)
    sel = jnp.where(
        lax.broadcasted_iota(jnp.int32, (_ROW_BLK, _L), 1) == tsub[:, None],
        bw_ref[...], 0.0)
    bias_blk = jnp.sum(sel, axis=1)
    out_ref[...] = bias_blk[:, None] + dot_scr[...]


def kernel(user_representations, targets, emb_table, bias_table):
    targets = targets.astype(jnp.int32)
    bwords = bias_table.reshape(bias_table.shape[0] // _L, _L)
    g, btiles = _sc_gather(targets, emb_table, bwords)
    tgt3d = targets.reshape(_B // _ROW_BLK, 1, _ROW_BLK)
    return pl.pallas_call(
        _bcast_body,
        grid=(_B // _ROW_BLK,),
        in_specs=[
            pl.BlockSpec((1, 1, _ROW_BLK), lambda i: (i, 0, 0)),
            pl.BlockSpec((_ROW_BLK, _L), lambda i: (i, 0)),
            pl.BlockSpec((_B, _D), lambda i: (0, 0)),
            pl.BlockSpec((_B, _D), lambda i: (0, 0)),
        ],
        out_specs=pl.BlockSpec((_ROW_BLK, _B), lambda i: (i, 0)),
        out_shape=jax.ShapeDtypeStruct((_B, _B), jnp.float32),
        scratch_shapes=[pltpu.VMEM((1, _B), jnp.float32)],
    )(tgt3d, btiles, user_representations, g)


# speedup vs baseline: 1.0056x; 1.0033x over previous
import functools

import jax
import jax.numpy as jnp
from jax import lax
from jax.experimental import pallas as pl
from jax.experimental.pallas import tpu as pltpu
from jax.experimental.pallas import tpu_sc as plsc

_B = 4096
_D = 64
_ROW_BLK = 256
_L = 16


def _sc_gather_lines(targets, emb2):
    info = plsc.get_sparse_core_info()
    nc, ns = info.num_cores, info.num_subcores
    bpw = _B // (nc * ns)

    mesh = plsc.VectorSubcoreMesh(core_axis_name="c", subcore_axis_name="s")

    @functools.partial(
        pl.kernel,
        mesh=mesh,
        compiler_params=pltpu.CompilerParams(
            use_tc_tiling_on_sc=True, needs_layout_passes=False),
        out_type=jax.ShapeDtypeStruct((_B, 2 * _D), jnp.float32),
        scratch_types=[
            pltpu.VMEM((bpw,), jnp.int32),
            pltpu.VMEM((bpw,), jnp.int32),
            pltpu.VMEM((bpw, 2 * _D), jnp.float32),
            pltpu.SemaphoreType.DMA,
        ],
    )
    def gather_kernel(tgt_hbm, emb2_hbm, lines_out, idx_v, line_v, lines_v, sem):
        wid = lax.axis_index("s") * nc + lax.axis_index("c")
        base = wid * bpw
        pltpu.sync_copy(tgt_hbm.at[pl.ds(base, bpw)], idx_v)
        for k in range(bpw // _L):
            sl = pl.ds(k * _L, _L)
            line_v[sl] = lax.shift_right_logical(idx_v[sl], 1)
        pltpu.async_copy(emb2_hbm.at[line_v], lines_v, sem).wait()
        pltpu.sync_copy(lines_v, lines_out.at[pl.ds(base, bpw)])

    return gather_kernel(targets, emb2)


def _sc_gather_bias(targets, bwords):
    info = plsc.get_sparse_core_info()
    nc, ns = info.num_cores, info.num_subcores
    bpw = _B // (nc * ns)

    mesh = plsc.VectorSubcoreMesh(core_axis_name="c", subcore_axis_name="s")

    @functools.partial(
        pl.kernel,
        mesh=mesh,
        compiler_params=pltpu.CompilerParams(
            use_tc_tiling_on_sc=False, needs_layout_passes=False),
        out_type=jax.ShapeDtypeStruct((_B, _L), jnp.float32),
        scratch_types=[
            pltpu.VMEM((bpw,), jnp.int32),
            pltpu.VMEM((bpw,), jnp.int32),
            pltpu.VMEM((bpw, _L), jnp.float32),
            pltpu.SemaphoreType.DMA,
        ],
    )
    def gather_kernel(tgt_hbm, bwords_hbm, bw_out, idx_v, word_v, bw_v, sem):
        wid = lax.axis_index("s") * nc + lax.axis_index("c")
        base = wid * bpw
        pltpu.sync_copy(tgt_hbm.at[pl.ds(base, bpw)], idx_v)
        for k in range(bpw // _L):
            sl = pl.ds(k * _L, _L)
            word_v[sl] = lax.shift_right_logical(idx_v[sl], 4)
        pltpu.async_copy(bwords_hbm.at[word_v], bw_v, sem).wait()
        pltpu.sync_copy(bw_v, bw_out.at[pl.ds(base, bpw)])

    return gather_kernel(targets, bwords)


def _bcast_body(tb_ref, bw_ref, tfull_ref, u_ref, g2_ref, out_ref, dot_scr):
    @pl.when(pl.program_id(0) == 0)
    def _():
        u = u_ref[...]
        dot_lo = jnp.sum(u * g2_ref[:, :_D], axis=1)[None, :]
        dot_hi = jnp.sum(u * g2_ref[:, _D:], axis=1)[None, :]
        par = tfull_ref[...] & 1
        dot_scr[...] = jnp.where(par == 1, dot_hi, dot_lo)

    tsub = tb_ref[...].reshape(_ROW_BLK) & (_L - 1)
    sel = jnp.where(
        lax.broadcasted_iota(jnp.int32, (_ROW_BLK, _L), 1) == tsub[:, None],
        bw_ref[...], 0.0)
    bias_blk = jnp.sum(sel, axis=1)
    out_ref[...] = bias_blk[:, None] + dot_scr[...]


def kernel(user_representations, targets, emb_table, bias_table):
    targets = targets.astype(jnp.int32)
    emb2 = emb_table.reshape(emb_table.shape[0] // 2, 2 * _D)
    bwords = bias_table.reshape(bias_table.shape[0] // _L, _L)
    g2 = _sc_gather_lines(targets, emb2)
    btiles = _sc_gather_bias(targets, bwords)
    tgt3d = targets.reshape(_B // _ROW_BLK, 1, _ROW_BLK)
    return pl.pallas_call(
        _bcast_body,
        grid=(_B // _ROW_BLK,),
        in_specs=[
            pl.BlockSpec((1, 1, _ROW_BLK), lambda i: (i, 0, 0)),
            pl.BlockSpec((_ROW_BLK, _L), lambda i: (i, 0)),
            pl.BlockSpec((1, _B), lambda i: (0, 0)),
            pl.BlockSpec((_B, _D), lambda i: (0, 0)),
            pl.BlockSpec((_B, 2 * _D), lambda i: (0, 0)),
        ],
        out_specs=pl.BlockSpec((_ROW_BLK, _B), lambda i: (i, 0)),
        out_shape=jax.ShapeDtypeStruct((_B, _B), jnp.float32),
        scratch_shapes=[pltpu.VMEM((1, _B), jnp.float32)],
    )(tgt3d, btiles, targets.reshape(1, _B), user_representations, g2)


# speedup vs baseline: 1.3368x; 1.3294x over previous
import functools

import jax
import jax.numpy as jnp
from jax import lax
from jax.experimental import pallas as pl
from jax.experimental.pallas import tpu as pltpu
from jax.experimental.pallas import tpu_sc as plsc

_B = 4096
_D = 64
_ROW_BLK = 256
_L = 16


def _sc_gather_rows(targets, emb_table):
    info = plsc.get_sparse_core_info()
    nc = info.num_cores
    bpw = _B // nc

    mesh = plsc.ScalarSubcoreMesh(axis_name="c", num_cores=nc)

    @functools.partial(
        pl.kernel,
        mesh=mesh,
        compiler_params=pltpu.CompilerParams(
            use_tc_tiling_on_sc=True, needs_layout_passes=False),
        out_type=jax.ShapeDtypeStruct((_B, _D), jnp.float32),
        scratch_types=[
            pltpu.SMEM((bpw,), jnp.int32),
            pltpu.SemaphoreType.DMA,
        ],
    )
    def gather_kernel(tgt_hbm, emb_hbm, rows_out, idx_s, sem):
        base = lax.axis_index("c") * bpw
        pltpu.sync_copy(tgt_hbm.at[pl.ds(base, bpw)], idx_s)

        @pl.loop(0, bpw)
        def _(j):
            pltpu.async_copy(
                emb_hbm.at[pl.ds(idx_s[j], 1)],
                rows_out.at[pl.ds(base + j, 1)], sem)

        pltpu.make_async_copy(
            emb_hbm.at[pl.ds(0, bpw)],
            rows_out.at[pl.ds(base, bpw)], sem).wait()

    return gather_kernel(targets, emb_table)


def _sc_gather_bias(targets, bwords):
    info = plsc.get_sparse_core_info()
    nc, ns = info.num_cores, info.num_subcores
    bpw = _B // (nc * ns)

    mesh = plsc.VectorSubcoreMesh(core_axis_name="c", subcore_axis_name="s")

    @functools.partial(
        pl.kernel,
        mesh=mesh,
        compiler_params=pltpu.CompilerParams(
            use_tc_tiling_on_sc=False, needs_layout_passes=False),
        out_type=jax.ShapeDtypeStruct((_B, _L), jnp.float32),
        scratch_types=[
            pltpu.VMEM((bpw,), jnp.int32),
            pltpu.VMEM((bpw,), jnp.int32),
            pltpu.VMEM((bpw, _L), jnp.float32),
            pltpu.SemaphoreType.DMA,
        ],
    )
    def gather_kernel(tgt_hbm, bwords_hbm, bw_out, idx_v, word_v, bw_v, sem):
        wid = lax.axis_index("s") * nc + lax.axis_index("c")
        base = wid * bpw
        pltpu.sync_copy(tgt_hbm.at[pl.ds(base, bpw)], idx_v)
        for k in range(bpw // _L):
            sl = pl.ds(k * _L, _L)
            word_v[sl] = lax.shift_right_logical(idx_v[sl], 4)
        pltpu.async_copy(bwords_hbm.at[word_v], bw_v, sem).wait()
        pltpu.sync_copy(bw_v, bw_out.at[pl.ds(base, bpw)])

    return gather_kernel(targets, bwords)


def _bcast_body(tb_ref, bw_ref, u_ref, g_ref, out_ref, dot_scr):
    @pl.when(pl.program_id(0) == 0)
    def _():
        dot_scr[...] = jnp.sum(u_ref[...] * g_ref[...], axis=1)[None, :]

    tsub = tb_ref[...].reshape(_ROW_BLK) & (_L - 1)
    sel = jnp.where(
        lax.broadcasted_iota(jnp.int32, (_ROW_BLK, _L), 1) == tsub[:, None],
        bw_ref[...], 0.0)
    bias_blk = jnp.sum(sel, axis=1)
    out_ref[...] = bias_blk[:, None] + dot_scr[...]


def kernel(user_representations, targets, emb_table, bias_table):
    targets = targets.astype(jnp.int32)
    bwords = bias_table.reshape(bias_table.shape[0] // _L, _L)
    g = _sc_gather_rows(targets, emb_table)
    btiles = _sc_gather_bias(targets, bwords)
    tgt3d = targets.reshape(_B // _ROW_BLK, 1, _ROW_BLK)
    return pl.pallas_call(
        _bcast_body,
        grid=(_B // _ROW_BLK,),
        in_specs=[
            pl.BlockSpec((1, 1, _ROW_BLK), lambda i: (i, 0, 0)),
            pl.BlockSpec((_ROW_BLK, _L), lambda i: (i, 0)),
            pl.BlockSpec((_B, _D), lambda i: (0, 0)),
            pl.BlockSpec((_B, _D), lambda i: (0, 0)),
        ],
        out_specs=pl.BlockSpec((_ROW_BLK, _B), lambda i: (i, 0)),
        out_shape=jax.ShapeDtypeStruct((_B, _B), jnp.float32),
        scratch_shapes=[pltpu.VMEM((1, _B), jnp.float32)],
    )(tgt3d, btiles, user_representations, g)
